# baseline (speedup 1.0000x reference)
import functools

import jax
import jax.numpy as jnp
from jax.experimental import pallas as pl
from jax.experimental.pallas import tpu as pltpu

BS = 8
NP = 16384
CIN = 128
COUT = 128
BLK = 256
NB = NP // BLK

NEG = float("-inf")


def _head(acc_fea, acc_geom, k, reps, W1, b1, W2, b2, out_ref):
    x = acc_fea
    h = jax.lax.dot(x, W1[0:CIN, :], preferred_element_type=jnp.float32)
    tail = W1[CIN:, :]
    for r in range(k * reps):
        h = h + acc_geom[:, (r % k):(r % k) + 1] * tail[r:r + 1, :]
    h = jax.nn.relu(h + b1[...])
    h = jax.nn.relu(
        jax.lax.dot(h, W2[...], preferred_element_type=jnp.float32) + b2[...])
    out_ref[...] = h


def _body(mad_f, adj_f, pt_f, cst_f, madg, adjg, ptg, xyzg,
          Wm1, bm1, Wm2, bm2, Wa1, ba1, Wa2, ba2,
          Wp1, bp1, Wp2, bp2, Wc1, bc1, Wc2, bc2,
          o_mad, o_adj, o_pt, o_cst,
          a_mad, a_adj, a_pt, a_cst, g_mad, g_adj, g_pt, g_cst):
    j = pl.program_id(0)

    fm = jnp.max(mad_f[...], axis=1)
    fa = jnp.max(adj_f[...], axis=1)
    fp = jnp.max(pt_f[...], axis=1)
    fc = jnp.max(cst_f[...], axis=1)
    gm = jnp.max(madg[...], axis=1)
    ga = jnp.max(adjg[...], axis=1)
    gp = jnp.max(ptg[...], axis=1)
    gc = jnp.max(xyzg[...], axis=1)

    @pl.when(j == 0)
    def _():
        a_mad[...] = fm
        a_adj[...] = fa
        a_pt[...] = fp
        a_cst[...] = fc
        g_mad[...] = gm
        g_adj[...] = ga
        g_pt[...] = gp
        g_cst[...] = gc

    @pl.when(j > 0)
    def _():
        a_mad[...] = jnp.maximum(a_mad[...], fm)
        a_adj[...] = jnp.maximum(a_adj[...], fa)
        a_pt[...] = jnp.maximum(a_pt[...], fp)
        a_cst[...] = jnp.maximum(a_cst[...], fc)
        g_mad[...] = jnp.maximum(g_mad[...], gm)
        g_adj[...] = jnp.maximum(g_adj[...], ga)
        g_pt[...] = jnp.maximum(g_pt[...], gp)
        g_cst[...] = jnp.maximum(g_cst[...], gc)

    @pl.when(j == NB - 1)
    def _():
        _head(a_mad[...], g_mad[...], 3, 1, Wm1, bm1, Wm2, bm2, o_mad)
        _head(a_adj[...], g_adj[...], 2, 2, Wa1, ba1, Wa2, ba2, o_adj)
        _head(a_pt[...], g_pt[...], 4, 2, Wp1, bp1, Wp2, bp2, o_pt)
        _head(a_cst[...], g_cst[...], 3, 1, Wc1, bc1, Wc2, bc2, o_cst)


def kernel(xyz, mad, adj, pt, mad_fea, adj_fea, pt_fea, cst_fea,
           W_mad1, b_mad1, W_mad2, b_mad2,
           W_adj1, b_adj1, W_adj2, b_adj2,
           W_pt1, b_pt1, W_pt2, b_pt2,
           W_cst1, b_cst1, W_cst2, b_cst2):
    fea_spec = pl.BlockSpec((BS, BLK, CIN), lambda j: (0, j, 0))

    def geo_spec(k):
        return pl.BlockSpec((BS, BLK, k), lambda j: (0, j, 0))

    def full(a):
        return pl.BlockSpec(a.shape, lambda j: (0,) * a.ndim)

    b_mad1, b_mad2 = b_mad1.reshape(1, -1), b_mad2.reshape(1, -1)
    b_adj1, b_adj2 = b_adj1.reshape(1, -1), b_adj2.reshape(1, -1)
    b_pt1, b_pt2 = b_pt1.reshape(1, -1), b_pt2.reshape(1, -1)
    b_cst1, b_cst2 = b_cst1.reshape(1, -1), b_cst2.reshape(1, -1)

    weights = (W_mad1, b_mad1, W_mad2, b_mad2,
               W_adj1, b_adj1, W_adj2, b_adj2,
               W_pt1, b_pt1, W_pt2, b_pt2,
               W_cst1, b_cst1, W_cst2, b_cst2)

    out_shape = [jax.ShapeDtypeStruct((BS, COUT), jnp.float32)] * 4
    out_spec = pl.BlockSpec((BS, COUT), lambda j: (0, 0))

    res = pl.pallas_call(
        _body,
        grid=(NB,),
        in_specs=[fea_spec] * 4
        + [geo_spec(3), geo_spec(2), geo_spec(4), geo_spec(3)]
        + [full(w) for w in weights],
        out_specs=[out_spec] * 4,
        out_shape=out_shape,
        scratch_shapes=[pltpu.VMEM((BS, COUT), jnp.float32)] * 4
        + [pltpu.VMEM((BS, 3), jnp.float32),
           pltpu.VMEM((BS, 2), jnp.float32),
           pltpu.VMEM((BS, 4), jnp.float32),
           pltpu.VMEM((BS, 3), jnp.float32)],
        compiler_params=pltpu.CompilerParams(
            dimension_semantics=("arbitrary",)),
    )(mad_fea, adj_fea, pt_fea, cst_fea, mad, adj, pt, xyz, *weights)
    return tuple(res)


# speedup vs baseline: 3.1670x; 3.1670x over previous
import functools

import jax
import jax.numpy as jnp
from jax.experimental import pallas as pl
from jax.experimental.pallas import tpu as pltpu

BS = 8
NP = 16384
CIN = 128
COUT = 128
BLK = 512
NB = NP // BLK

NEG = float("-inf")


def _head(acc_fea, acc_geom, k, reps, W1, b1, W2, b2, out_ref):
    x = acc_fea
    h = jax.lax.dot(x, W1[0:CIN, :], preferred_element_type=jnp.float32)
    tail = W1[CIN:, :]
    for r in range(k * reps):
        h = h + acc_geom[:, (r % k):(r % k) + 1] * tail[r:r + 1, :]
    h = jax.nn.relu(h + b1[...])
    h = jax.nn.relu(
        jax.lax.dot(h, W2[...], preferred_element_type=jnp.float32) + b2[...])
    out_ref[...] = h


def _body(mad_f, adj_f, pt_f, cst_f, madg, adjg, ptg, xyzg,
          Wm1, bm1, Wm2, bm2, Wa1, ba1, Wa2, ba2,
          Wp1, bp1, Wp2, bp2, Wc1, bc1, Wc2, bc2,
          o_mad, o_adj, o_pt, o_cst,
          a_mad, a_adj, a_pt, a_cst, g_mad, g_adj, g_pt, g_cst):
    j = pl.program_id(0)

    fm = jnp.max(mad_f[...], axis=1)
    fa = jnp.max(adj_f[...], axis=1)
    fp = jnp.max(pt_f[...], axis=1)
    fc = jnp.max(cst_f[...], axis=1)
    gm = jnp.max(madg[...], axis=2)
    ga = jnp.max(adjg[...], axis=2)
    gp = jnp.max(ptg[...], axis=2)
    gc = jnp.max(xyzg[...], axis=2)

    @pl.when(j == 0)
    def _():
        a_mad[...] = fm
        a_adj[...] = fa
        a_pt[...] = fp
        a_cst[...] = fc
        g_mad[...] = gm
        g_adj[...] = ga
        g_pt[...] = gp
        g_cst[...] = gc

    @pl.when(j > 0)
    def _():
        a_mad[...] = jnp.maximum(a_mad[...], fm)
        a_adj[...] = jnp.maximum(a_adj[...], fa)
        a_pt[...] = jnp.maximum(a_pt[...], fp)
        a_cst[...] = jnp.maximum(a_cst[...], fc)
        g_mad[...] = jnp.maximum(g_mad[...], gm)
        g_adj[...] = jnp.maximum(g_adj[...], ga)
        g_pt[...] = jnp.maximum(g_pt[...], gp)
        g_cst[...] = jnp.maximum(g_cst[...], gc)

    @pl.when(j == NB - 1)
    def _():
        _head(a_mad[...], g_mad[...], 3, 1, Wm1, bm1, Wm2, bm2, o_mad)
        _head(a_adj[...], g_adj[...], 2, 2, Wa1, ba1, Wa2, ba2, o_adj)
        _head(a_pt[...], g_pt[...], 4, 2, Wp1, bp1, Wp2, bp2, o_pt)
        _head(a_cst[...], g_cst[...], 3, 1, Wc1, bc1, Wc2, bc2, o_cst)


def kernel(xyz, mad, adj, pt, mad_fea, adj_fea, pt_fea, cst_fea,
           W_mad1, b_mad1, W_mad2, b_mad2,
           W_adj1, b_adj1, W_adj2, b_adj2,
           W_pt1, b_pt1, W_pt2, b_pt2,
           W_cst1, b_cst1, W_cst2, b_cst2):
    fea_spec = pl.BlockSpec((BS, BLK, CIN), lambda j: (0, j, 0))

    def geo_spec(k):
        return pl.BlockSpec((BS, k, BLK), lambda j: (0, 0, j))

    def full(a):
        return pl.BlockSpec(a.shape, lambda j: (0,) * a.ndim)

    b_mad1, b_mad2 = b_mad1.reshape(1, -1), b_mad2.reshape(1, -1)
    b_adj1, b_adj2 = b_adj1.reshape(1, -1), b_adj2.reshape(1, -1)
    b_pt1, b_pt2 = b_pt1.reshape(1, -1), b_pt2.reshape(1, -1)
    b_cst1, b_cst2 = b_cst1.reshape(1, -1), b_cst2.reshape(1, -1)

    weights = (W_mad1, b_mad1, W_mad2, b_mad2,
               W_adj1, b_adj1, W_adj2, b_adj2,
               W_pt1, b_pt1, W_pt2, b_pt2,
               W_cst1, b_cst1, W_cst2, b_cst2)

    out_shape = [jax.ShapeDtypeStruct((BS, COUT), jnp.float32)] * 4
    out_spec = pl.BlockSpec((BS, COUT), lambda j: (0, 0))

    res = pl.pallas_call(
        _body,
        grid=(NB,),
        in_specs=[fea_spec] * 4
        + [geo_spec(3), geo_spec(2), geo_spec(4), geo_spec(3)]
        + [full(w) for w in weights],
        out_specs=[out_spec] * 4,
        out_shape=out_shape,
        scratch_shapes=[pltpu.VMEM((BS, COUT), jnp.float32)] * 4
        + [pltpu.VMEM((BS, 3), jnp.float32),
           pltpu.VMEM((BS, 2), jnp.float32),
           pltpu.VMEM((BS, 4), jnp.float32),
           pltpu.VMEM((BS, 3), jnp.float32)],
        compiler_params=pltpu.CompilerParams(
            dimension_semantics=("arbitrary",)),
    )(mad_fea, adj_fea, pt_fea, cst_fea,
      mad.transpose(0, 2, 1), adj.transpose(0, 2, 1),
      pt.transpose(0, 2, 1), xyz.transpose(0, 2, 1), *weights)
    return tuple(res)
